# baseline (speedup 1.0000x reference)
import functools

import jax
import jax.numpy as jnp
from jax import lax
from jax.experimental import pallas as pl
from jax.experimental.pallas import tpu as pltpu
from jax.experimental.pallas import tpu_sc as plsc

_N, _C, _D, _H, _W = 4, 128, 2, 200, 176
_CD = _C * _D
_HW = _H * _W
_NNZ = 150000
_NC = 2
_NS = 16
_NW = _NC * _NS
_HB = _H // (_NW // _N)
_RW = _HB * _W
_CHUNK = 6000
_NCHUNK = _NNZ // _CHUNK
_QPC = _CHUNK // 16
_AG = 15
_GSUB = 88
_NU = 2 * _HB
_MAPN = 2 * _RW


def _hc_body(feat_hbm, idx_hbm, out_hbm, map_v, idxa, idxb,
             rows_a, rows_b, ob_a, ob_b,
             sem_ia, sem_ib, sem_ga, sem_gb, sem_oa, sem_ob):
    wid = lax.axis_index("s") * _NC + lax.axis_index("c")
    n = wid // (_NW // _N)
    hb = wid - n * (_NW // _N)
    base0 = n * (_D * _HW) + hb * _RW
    base1 = base0 + _HW
    iota16 = lax.broadcasted_iota(jnp.int32, (16,), 0)

    def init_body(i, _):
        map_v[pl.ds(i * 16, 16)] = jnp.full((16,), -1, jnp.int32)
        return 0

    lax.fori_loop(0, _MAPN // 16, init_body, 0)

    def idx_copy(c, buf, sem):
        return pltpu.make_async_copy(
            idx_hbm.at[pl.ds(c * _CHUNK, _CHUNK)], buf, sem)

    def scan_chunk(c, buf):
        def q_body(q, _):
            gs = [buf[pl.ds((q * _AG + t) * 16, 16)] for t in range(_AG)]
            for t in range(_AG):
                g = gs[t]
                s0 = g - base0
                s1 = g - base1
                in0 = s0.astype(jnp.uint32) < jnp.uint32(_RW)
                in1 = s1.astype(jnp.uint32) < jnp.uint32(_RW)
                m = in0 | in1
                local = jnp.where(in0, s0, s1 + _RW)
                local = jnp.where(m, local, 0)
                vid = (c * _CHUNK + (q * _AG + t) * 16) + iota16
                plsc.store_scatter(map_v, [local], vid, mask=m)
            return 0

        lax.fori_loop(0, _QPC // _AG, q_body, 0)

    idx_copy(0, idxa, sem_ia).start()

    def a_body(t, _):
        a = 2 * t
        b = a + 1

        @pl.when(b < _NCHUNK)
        def _():
            idx_copy(b, idxb, sem_ib).start()

        idx_copy(a, idxa, sem_ia).wait()
        scan_chunk(a, idxa)

        @pl.when(a + 2 < _NCHUNK)
        def _():
            idx_copy(a + 2, idxa, sem_ia).start()

        @pl.when(b < _NCHUNK)
        def _():
            idx_copy(b, idxb, sem_ib).wait()
            scan_chunk(b, idxb)

        return 0

    lax.fori_loop(0, (_NCHUNK + 1) // 2, a_body, 0)

    ch_idx = [[(i * 16 + iota16) * _D + d for i in range(_C // 16)]
              for d in range(_D)]

    zero16 = jnp.zeros((16,), jnp.float32)

    def zero_rows(rows):
        @plsc.parallel_loop(0, _GSUB, 1, unroll=2)
        def z_body(w):
            for half in range(2):
                for i in range(_C // 16):
                    rows[_GSUB * half + w, pl.ds(i * 16, 16)] = zero16

    def gathers(r, w0, rows, sem):
        mo = r * _W + w0
        cps = []
        for d in range(2):
            cp = pltpu.make_async_copy(
                feat_hbm.at[plsc.Indices(map_v.at[pl.ds(d * _RW + mo, _GSUB)],
                                         ignored_value=-1)],
                rows.at[pl.ds(d * _GSUB, _GSUB)], sem)
            cps.append(cp)
        return cps

    def out_copy(r, w0, ob, sem):
        return pltpu.make_async_copy(
            ob, out_hbm.at[n, hb * _HB + r, pl.ds(w0, _GSUB), :], sem)

    def interleave(rows, ob):
        @plsc.parallel_loop(0, _GSUB, 1, unroll=2)
        def w_body(w):
            wv = jnp.full((16,), 0, jnp.int32) + w
            for i in range(_C // 16):
                v0 = rows[w, pl.ds(i * 16, 16)]
                plsc.store_scatter(ob, [wv, ch_idx[0][i]], v0)
                v1 = rows[_GSUB + w, pl.ds(i * 16, 16)]
                plsc.store_scatter(ob, [wv, ch_idx[1][i]], v1)

    zero_rows(rows_a)
    for cp in gathers(0, 0, rows_a, sem_ga):
        cp.start()

    def b_body(t, _):
        zero_rows(rows_b)
        for cp in gathers(t, _GSUB, rows_b, sem_gb):
            cp.start()
        for cp in gathers(t, 0, rows_a, sem_ga):
            cp.wait()

        @pl.when(t > 0)
        def _():
            out_copy(t, 0, ob_a, sem_oa).wait()

        interleave(rows_a, ob_a)
        out_copy(t, 0, ob_a, sem_oa).start()

        @pl.when(t + 1 < _HB)
        def _():
            zero_rows(rows_a)
            for cp in gathers(t + 1, 0, rows_a, sem_ga):
                cp.start()

        for cp in gathers(t, _GSUB, rows_b, sem_gb):
            cp.wait()

        @pl.when(t > 0)
        def _():
            out_copy(t, _GSUB, ob_b, sem_ob).wait()

        interleave(rows_b, ob_b)
        out_copy(t, _GSUB, ob_b, sem_ob).start()
        return 0

    lax.fori_loop(0, _HB, b_body, 0)
    out_copy(_HB - 1, 0, ob_a, sem_oa).wait()
    out_copy(_HB - 1, _GSUB, ob_b, sem_ob).wait()


_hc_kernel = functools.partial(
    pl.kernel,
    out_type=jax.ShapeDtypeStruct((_N, _H, _W, _CD), jnp.float32),
    mesh=plsc.VectorSubcoreMesh(core_axis_name="c", subcore_axis_name="s"),
    scratch_types=[
        pltpu.VMEM((_MAPN,), jnp.int32),
        pltpu.VMEM((_CHUNK,), jnp.int32),
        pltpu.VMEM((_CHUNK,), jnp.int32),
        pltpu.VMEM((_D * _GSUB, _C), jnp.float32),
        pltpu.VMEM((_D * _GSUB, _C), jnp.float32),
        pltpu.VMEM((_GSUB, _CD), jnp.float32),
        pltpu.VMEM((_GSUB, _CD), jnp.float32),
        pltpu.SemaphoreType.DMA,
        pltpu.SemaphoreType.DMA,
        pltpu.SemaphoreType.DMA,
        pltpu.SemaphoreType.DMA,
        pltpu.SemaphoreType.DMA,
        pltpu.SemaphoreType.DMA,
    ],
    compiler_params=pltpu.CompilerParams(
        needs_layout_passes=False, use_tc_tiling_on_sc=True),
)(_hc_body)


def kernel(features, indices):
    out = _hc_kernel(features, indices)
    return jnp.transpose(out, (0, 3, 1, 2))


# speedup vs baseline: 1.0005x; 1.0005x over previous
import functools

import jax
import jax.numpy as jnp
from jax import lax
from jax.experimental import pallas as pl
from jax.experimental.pallas import tpu as pltpu
from jax.experimental.pallas import tpu_sc as plsc

_N, _C, _D, _H, _W = 4, 128, 2, 200, 176
_CD = _C * _D
_HW = _H * _W
_NNZ = 150000
_NC = 2
_NS = 16
_NW = _NC * _NS
_HB = _H // (_NW // _N)
_RW = _HB * _W
_CHUNK = 6000
_NCHUNK = _NNZ // _CHUNK
_QPC = _CHUNK // 16
_AG = 15
_GSUB = 88
_MAPN = 2 * _RW


def _hc_body(feat_hbm, idx_hbm, out_hbm, map_v, idxa, idxb,
             rows_a, rows_b, ob_a, ob_b,
             sem_ia, sem_ib, sem_ga, sem_gb, sem_oa, sem_ob):
    wid = lax.axis_index("s") * _NC + lax.axis_index("c")
    n = wid // (_NW // _N)
    hb = wid - n * (_NW // _N)
    base0 = n * (_D * _HW) + hb * _RW
    base1 = base0 + _HW
    iota16 = lax.broadcasted_iota(jnp.int32, (16,), 0)

    def init_body(i, _):
        map_v[pl.ds(i * 16, 16)] = jnp.full((16,), -1, jnp.int32)
        return 0

    lax.fori_loop(0, _MAPN // 16, init_body, 0)

    def idx_copy(c, buf, sem):
        return pltpu.make_async_copy(
            idx_hbm.at[pl.ds(c * _CHUNK, _CHUNK)], buf, sem)

    def scan_chunk(c, buf):
        def q_body(q, _):
            gs = [buf[pl.ds((q * _AG + t) * 16, 16)] for t in range(_AG)]
            for t in range(_AG):
                g = gs[t]
                s0 = g - base0
                s1 = g - base1
                in0 = s0.astype(jnp.uint32) < jnp.uint32(_RW)
                in1 = s1.astype(jnp.uint32) < jnp.uint32(_RW)
                m = in0 | in1
                local = jnp.where(in0, s0, s1 + _RW)
                local = jnp.where(m, local, 0)
                vid = (c * _CHUNK + (q * _AG + t) * 16) + iota16
                plsc.store_scatter(map_v, [local], vid, mask=m)
            return 0

        lax.fori_loop(0, _QPC // _AG, q_body, 0)

    idx_copy(0, idxa, sem_ia).start()

    def a_body(t, _):
        a = 2 * t
        b = a + 1

        @pl.when(b < _NCHUNK)
        def _():
            idx_copy(b, idxb, sem_ib).start()

        idx_copy(a, idxa, sem_ia).wait()
        scan_chunk(a, idxa)

        @pl.when(a + 2 < _NCHUNK)
        def _():
            idx_copy(a + 2, idxa, sem_ia).start()

        @pl.when(b < _NCHUNK)
        def _():
            idx_copy(b, idxb, sem_ib).wait()
            scan_chunk(b, idxb)

        return 0

    lax.fori_loop(0, (_NCHUNK + 1) // 2, a_body, 0)

    ch_idx = [[(i * 16 + iota16) * _D + d for i in range(_C // 16)]
              for d in range(_D)]

    zero16 = jnp.zeros((16,), jnp.float32)

    def zero_rows(rows):
        @plsc.parallel_loop(0, _GSUB, 1, unroll=2)
        def z_body(w):
            for half in range(2):
                for i in range(_C // 16):
                    rows[_GSUB * half + w, pl.ds(i * 16, 16)] = zero16

    def gathers(r, w0, rows, sem):
        mo = r * _W + w0
        cps = []
        for d in range(2):
            cp = pltpu.make_async_copy(
                feat_hbm.at[plsc.Indices(map_v.at[pl.ds(d * _RW + mo, _GSUB)],
                                         ignored_value=-1)],
                rows.at[pl.ds(d * _GSUB, _GSUB)], sem)
            cps.append(cp)
        return cps

    def out_copy(r, w0, ob, sem):
        return pltpu.make_async_copy(
            ob, out_hbm.at[n, hb * _HB + r, pl.ds(w0, _GSUB), :], sem)

    def interleave(rows, ob):
        @plsc.parallel_loop(0, _GSUB, 1, unroll=2)
        def w_body(w):
            wv = jnp.full((16,), 0, jnp.int32) + w
            for i in range(_C // 16):
                v0 = rows[w, pl.ds(i * 16, 16)]
                plsc.store_scatter(ob, [wv, ch_idx[0][i]], v0)
                v1 = rows[_GSUB + w, pl.ds(i * 16, 16)]
                plsc.store_scatter(ob, [wv, ch_idx[1][i]], v1)

    zero_rows(rows_a)
    for cp in gathers(0, 0, rows_a, sem_ga):
        cp.start()

    def b_body(t, _):
        zero_rows(rows_b)
        for cp in gathers(t, _GSUB, rows_b, sem_gb):
            cp.start()
        for cp in gathers(t, 0, rows_a, sem_ga):
            cp.wait()

        @pl.when(t > 0)
        def _():
            out_copy(t, 0, ob_a, sem_oa).wait()

        interleave(rows_a, ob_a)
        out_copy(t, 0, ob_a, sem_oa).start()

        @pl.when(t + 1 < _HB)
        def _():
            zero_rows(rows_a)
            for cp in gathers(t + 1, 0, rows_a, sem_ga):
                cp.start()

        for cp in gathers(t, _GSUB, rows_b, sem_gb):
            cp.wait()

        @pl.when(t > 0)
        def _():
            out_copy(t, _GSUB, ob_b, sem_ob).wait()

        interleave(rows_b, ob_b)
        out_copy(t, _GSUB, ob_b, sem_ob).start()
        return 0

    lax.fori_loop(0, _HB, b_body, 0)
    out_copy(_HB - 1, 0, ob_a, sem_oa).wait()
    out_copy(_HB - 1, _GSUB, ob_b, sem_ob).wait()


_hc_kernel = functools.partial(
    pl.kernel,
    out_type=jax.ShapeDtypeStruct((_N, _H, _W, _CD), jnp.float32),
    mesh=plsc.VectorSubcoreMesh(core_axis_name="c", subcore_axis_name="s"),
    scratch_types=[
        pltpu.VMEM((_MAPN,), jnp.int32),
        pltpu.VMEM((_CHUNK,), jnp.int32),
        pltpu.VMEM((_CHUNK,), jnp.int32),
        pltpu.VMEM((_D * _GSUB, _C), jnp.float32),
        pltpu.VMEM((_D * _GSUB, _C), jnp.float32),
        pltpu.VMEM((_GSUB, _CD), jnp.float32),
        pltpu.VMEM((_GSUB, _CD), jnp.float32),
        pltpu.SemaphoreType.DMA,
        pltpu.SemaphoreType.DMA,
        pltpu.SemaphoreType.DMA,
        pltpu.SemaphoreType.DMA,
        pltpu.SemaphoreType.DMA,
        pltpu.SemaphoreType.DMA,
    ],
    compiler_params=pltpu.CompilerParams(
        needs_layout_passes=False, use_tc_tiling_on_sc=True),
)(_hc_body)


def kernel(features, indices):
    out = _hc_kernel(features, indices)
    return jnp.transpose(out, (0, 3, 1, 2))
